# baseline (speedup 1.0000x reference)
import functools

import jax
import jax.numpy as jnp
from jax import lax
from jax.experimental import pallas as pl
from jax.experimental.pallas import tpu as pltpu
from jax.experimental.pallas import tpu_sc as plsc

N = 16384
EMB_DIM = 16
NUM_CORES = 2
NUM_SUBCORES = 16
LANES = 16
NUM_WORKERS = NUM_CORES * NUM_SUBCORES
ROWS_PER_WORKER = N // NUM_WORKERS
BLOCK = 128
CHUNK = 32


def _build():
    mesh = plsc.VectorSubcoreMesh(core_axis_name="c", subcore_axis_name="s")

    @functools.partial(
        pl.kernel,
        mesh=mesh,
        out_type=jax.ShapeDtypeStruct((N, EMB_DIM), jnp.float32),
        scratch_types=[
            pltpu.VMEM((ROWS_PER_WORKER,), jnp.int32),
            pltpu.VMEM((CHUNK * EMB_DIM, BLOCK), jnp.float32),
            pltpu.VMEM((CHUNK, EMB_DIM), jnp.float32),
            pltpu.VMEM((EMB_DIM * LANES,), jnp.float32),
            pltpu.SemaphoreType.DMA,
        ],
    )
    def gather_kernel(xt_hbm, wt_hbm, out_hbm, xv, bufs, slab, shift, sem):
        wid = lax.axis_index("s") * NUM_CORES + lax.axis_index("c")
        base = wid * ROWS_PER_WORKER

        pltpu.sync_copy(xt_hbm.at[0, pl.ds(base, ROWS_PER_WORKER)], xv)

        iota = lax.iota(jnp.int32, LANES)

        def chunk_body(c, carry0):
            def issue_body(t, _):
                v = xv[pl.ds(c * CHUNK + t * LANES, LANES)]
                for r in range(LANES):
                    lane0 = pl.multiple_of(
                        lax.bitwise_and(v[r], jnp.int32(-BLOCK)), BLOCK
                    )
                    k = t * LANES + r
                    pltpu.async_copy(
                        wt_hbm.at[:, pl.ds(lane0, BLOCK)],
                        bufs.at[pl.ds(k * EMB_DIM, EMB_DIM), :],
                        sem,
                    )
                return _

            lax.fori_loop(0, CHUNK // LANES, issue_body, None)

            def drain_body(t, _):
                for r in range(LANES):
                    k = t * LANES + r
                    pltpu.make_async_copy(
                        wt_hbm.at[:, pl.ds(0, BLOCK)],
                        bufs.at[pl.ds(k * EMB_DIM, EMB_DIM), :],
                        sem,
                    ).wait()
                return _

            lax.fori_loop(0, CHUNK // LANES, drain_body, None)

            def extract_body(t, carry):
                v = xv[pl.ds(c * CHUNK + t * LANES, LANES)]
                for r in range(LANES):
                    k = t * LANES + r
                    lane = lax.bitwise_and(v[r], jnp.int32(BLOCK - 1))
                    l16 = pl.multiple_of(
                        lax.bitwise_and(lane, jnp.int32(-LANES)), LANES
                    )
                    lsub = lax.bitwise_and(lane, jnp.int32(LANES - 1))
                    for d in range(EMB_DIM):
                        shift[pl.ds(d * LANES, LANES)] = bufs[
                            k * EMB_DIM + d, pl.ds(l16, LANES)
                        ]
                    acc = jnp.zeros((LANES,), jnp.float32)
                    for d in range(EMB_DIM):
                        r = shift[pl.ds(15 * d + lsub, LANES)]
                        acc = jnp.where(iota == d, r, acc)
                    slab[k, :] = acc
                return carry

            lax.fori_loop(0, CHUNK // LANES, extract_body, None)

            row0 = pl.multiple_of(base + c * CHUNK, CHUNK)
            pltpu.sync_copy(slab, out_hbm.at[pl.ds(row0, CHUNK)])
            return carry0

        lax.fori_loop(0, ROWS_PER_WORKER // CHUNK, chunk_body, None)

    return gather_kernel


_gather = _build()


def kernel(x, W):
    return _gather(x.T, W.T)


# speedup vs baseline: 1.2422x; 1.2422x over previous
import functools

import jax
import jax.numpy as jnp
from jax import lax
from jax.experimental import pallas as pl
from jax.experimental.pallas import tpu as pltpu
from jax.experimental.pallas import tpu_sc as plsc

N = 16384
EMB_DIM = 16
NUM_CORES = 2
NUM_SUBCORES = 16
LANES = 16
NUM_WORKERS = NUM_CORES * NUM_SUBCORES
ROWS_PER_WORKER = N // NUM_WORKERS
BLOCK = 128
CHUNK = 16
NCHUNKS = ROWS_PER_WORKER // CHUNK
BANK_ROWS = CHUNK * EMB_DIM


def _build():
    mesh = plsc.VectorSubcoreMesh(core_axis_name="c", subcore_axis_name="s")

    @functools.partial(
        pl.kernel,
        mesh=mesh,
        out_type=jax.ShapeDtypeStruct((N, EMB_DIM), jnp.float32),
        scratch_types=[
            pltpu.VMEM((ROWS_PER_WORKER,), jnp.int32),
            pltpu.VMEM((2 * BANK_ROWS, BLOCK), jnp.float32),
            pltpu.VMEM((CHUNK, EMB_DIM), jnp.float32),
            pltpu.VMEM((EMB_DIM * LANES,), jnp.float32),
            pltpu.SemaphoreType.DMA,
            pltpu.SemaphoreType.DMA,
        ],
    )
    def gather_kernel(xt_hbm, wt_hbm, out_hbm, xv, bufs, slab, shift, sem0,
                      sem1):
        wid = lax.axis_index("s") * NUM_CORES + lax.axis_index("c")
        base = wid * ROWS_PER_WORKER

        pltpu.sync_copy(xt_hbm.at[0, pl.ds(base, ROWS_PER_WORKER)], xv)

        iota = lax.iota(jnp.int32, LANES)
        sems = (sem0, sem1)

        def issue(c, bank):
            v = xv[pl.ds(c * CHUNK, CHUNK)]
            for r in range(CHUNK):
                lane0 = pl.multiple_of(
                    lax.bitwise_and(v[r], jnp.int32(-BLOCK)), BLOCK
                )
                row0 = bank * BANK_ROWS + r * EMB_DIM
                pltpu.async_copy(
                    wt_hbm.at[:, pl.ds(lane0, BLOCK)],
                    bufs.at[pl.ds(row0, EMB_DIM), :],
                    sems[bank],
                )

        def drain(bank):
            for r in range(CHUNK):
                row0 = bank * BANK_ROWS + r * EMB_DIM
                pltpu.make_async_copy(
                    wt_hbm.at[:, pl.ds(0, BLOCK)],
                    bufs.at[pl.ds(row0, EMB_DIM), :],
                    sems[bank],
                ).wait()

        def extract_and_write(c, bank):
            v = xv[pl.ds(c * CHUNK, CHUNK)]
            for r in range(CHUNK):
                lane = lax.bitwise_and(v[r], jnp.int32(BLOCK - 1))
                l16 = pl.multiple_of(
                    lax.bitwise_and(lane, jnp.int32(-LANES)), LANES
                )
                lsub = lax.bitwise_and(lane, jnp.int32(LANES - 1))
                row0 = bank * BANK_ROWS + r * EMB_DIM
                for d in range(EMB_DIM):
                    shift[pl.ds(d * LANES, LANES)] = bufs[
                        row0 + d, pl.ds(l16, LANES)
                    ]
                acc = jnp.zeros((LANES,), jnp.float32)
                for d in range(EMB_DIM):
                    rr = shift[pl.ds(15 * d + lsub, LANES)]
                    acc = jnp.where(iota == d, rr, acc)
                slab[r, :] = acc
            row_out = pl.multiple_of(base + c * CHUNK, CHUNK)
            pltpu.sync_copy(slab, out_hbm.at[pl.ds(row_out, CHUNK)])

        issue(jnp.int32(0), 0)

        def pair_body(c2, carry):
            c_even = 2 * c2
            c_odd = c_even + 1
            issue(c_odd, 1)
            drain(0)
            extract_and_write(c_even, 0)

            @pl.when(c_odd + 1 < NCHUNKS)
            def _issue_next():
                issue(c_odd + 1, 0)

            drain(1)
            extract_and_write(c_odd, 1)
            return carry

        lax.fori_loop(0, NCHUNKS // 2, pair_body, None)

    return gather_kernel


_gather = _build()


def kernel(x, W):
    return _gather(x.T, W.T)
